# baseline (speedup 1.0000x reference)
import functools

import jax
import jax.numpy as jnp
from jax import lax
from jax.experimental import pallas as pl
from jax.experimental.pallas import tpu as pltpu
from jax.experimental.pallas import tpu_sc as plsc

B = 8
P = 1250
PP = 1280
NP = B * PP
R = 128
K = 20
BIG = 1e30


def _ln(x, g, b):
    mu = jnp.mean(x, axis=-1, keepdims=True)
    v = jnp.mean((x - mu) ** 2, axis=-1, keepdims=True)
    return (x - mu) / jnp.sqrt(v + 1e-5) * g + b


def _bf(a):
    return a.astype(jnp.bfloat16)


def _knn_body(xg_ref, idx_ref):
    b = pl.program_id(0)
    r = pl.program_id(1)
    X = xg_ref[0]
    Xr = xg_ref[0, pl.ds(r * R, R), :]

    sq = jnp.sum(X * X, axis=1)
    sqr = jnp.sum(Xr * Xr, axis=1)
    cross = jnp.dot(_bf(Xr), _bf(X).T, preferred_element_type=jnp.float32)
    dist = sqr[:, None] - 2.0 * cross + sq[None, :]
    col = lax.broadcasted_iota(jnp.int32, (R, PP), 1)
    dist = jnp.where(col >= P, BIG, dist)

    base = b * PP
    for k in range(K):
        m = jnp.min(dist, axis=1, keepdims=True)
        hot = dist == m
        am = jnp.min(jnp.where(hot, col, PP), axis=1, keepdims=True)
        dist = jnp.where(col == am, BIG, dist)
        idx_ref[k, 0] = am + base


def _knn(xg, d):
    return pl.pallas_call(
        _knn_body,
        grid=(B, PP // R),
        compiler_params=pltpu.CompilerParams(
            dimension_semantics=("parallel", "parallel")),
        in_specs=[pl.BlockSpec((1, PP, d), lambda b, r: (b, 0, 0))],
        out_specs=pl.BlockSpec((K, 1, R, 1), lambda b, r: (0, b, r, 0)),
        out_shape=jax.ShapeDtypeStruct((K, B, PP, 1), jnp.int32),
    )(xg)


def _sc_gather(table, idx, d):
    info = plsc.get_sparse_core_info()
    nw = info.num_cores * info.num_subcores
    tot = K * NP
    b_per_w = tot // nw
    ch = 800
    n_ch = b_per_w // ch
    mesh = plsc.VectorSubcoreMesh(core_axis_name="c", subcore_axis_name="s")

    @functools.partial(
        pl.kernel, mesh=mesh,
        out_type=jax.ShapeDtypeStruct((tot, d), jnp.float32),
        compiler_params=pltpu.CompilerParams(use_tc_tiling_on_sc=False),
        scratch_types=[
            pltpu.VMEM((ch,), jnp.int32),
            pltpu.VMEM((ch, d), jnp.float32),
            pltpu.SemaphoreType.DMA,
        ],
    )
    def gk(table_hbm, idx_hbm, out_hbm, idx_v, rows_v, sem):
        wid = lax.axis_index("s") * info.num_cores + lax.axis_index("c")
        base = wid * b_per_w
        for c in range(n_ch):
            off = base + c * ch
            pltpu.sync_copy(idx_hbm.at[pl.ds(off, ch)], idx_v)
            pltpu.async_copy(table_hbm.at[idx_v], rows_v, sem).wait()
            pltpu.sync_copy(rows_v, out_hbm.at[pl.ds(off, ch)])

    return gk(table, idx)


def _edge_mlp_body(xg_ref, xj_ref, w1_ref, b1_ref, g1_ref, be1_ref,
                   w2_ref, b2_ref, g2_ref, be2_ref, out_ref):
    r = pl.program_id(1)
    xr = xg_ref[0]
    xjs = xj_ref[:, 0]

    xi = jnp.concatenate([xr] * K, axis=0)
    xj = jnp.concatenate([xjs[k] for k in range(K)], axis=0)
    msg = jnp.concatenate([xi, xj - xi], axis=1)

    h = jnp.dot(_bf(msg), _bf(w1_ref[...].T),
                preferred_element_type=jnp.float32) + b1_ref[...]
    h = jax.nn.relu(_ln(h, g1_ref[...], be1_ref[...]))
    h = jnp.dot(_bf(h), _bf(w2_ref[...].T),
                preferred_element_type=jnp.float32) + b2_ref[...]
    h = jax.nn.relu(_ln(h, g2_ref[...], be2_ref[...]))

    H = h.shape[1]
    acc = h[0:R]
    for k in range(1, K):
        acc = jnp.maximum(acc, h[k * R:(k + 1) * R])

    rowid = lax.broadcasted_iota(jnp.int32, (R, H), 0) + r * R
    out_ref[0] = jnp.where(rowid < P, acc, 0.0)


def _edge_mlp(xg, xj, p, d, h):
    w1, b1, g1, be1, w2, b2, g2, be2 = p
    full = lambda s: pl.BlockSpec(s, lambda b, r: (0, 0))
    return pl.pallas_call(
        _edge_mlp_body,
        grid=(B, PP // R),
        compiler_params=pltpu.CompilerParams(
            dimension_semantics=("parallel", "parallel")),
        in_specs=[
            pl.BlockSpec((1, R, d), lambda b, r: (b, r, 0)),
            pl.BlockSpec((K, 1, R, d), lambda b, r: (0, b, r, 0)),
            full((h, w1.shape[1])),
            full((1, h)), full((1, h)), full((1, h)),
            full((h, h)),
            full((1, h)), full((1, h)), full((1, h)),
        ],
        out_specs=pl.BlockSpec((1, R, h), lambda b, r: (b, r, 0)),
        out_shape=jax.ShapeDtypeStruct((B, PP, h), jnp.float32),
    )(xg, xj, w1,
      b1.reshape(1, h), g1.reshape(1, h), be1.reshape(1, h),
      w2, b2.reshape(1, h), g2.reshape(1, h), be2.reshape(1, h))


def _edge_conv(xg, p, d, h, w1pad=None):
    idx = _knn(xg, d)
    rows = _sc_gather(xg.reshape(NP, d), idx.reshape(K * NP), d)
    xj = rows.reshape(K, B, PP, d)
    w1 = p[0] if w1pad is None else w1pad
    return _edge_mlp(xg, xj, (w1,) + tuple(p[1:]), d, h)


def _pool_glob_body(x1_ref, x2_ref, x3_ref,
                    gw1_ref, gb1_ref, gg1_ref, gbe1_ref,
                    gw2_ref, gb2_ref, gg2_ref, gbe2_ref, g_ref):
    rows = []
    for b in range(B):
        loc = jnp.concatenate([x1_ref[b], x2_ref[b], x3_ref[b]], axis=1)
        rows.append(jnp.max(loc, axis=0, keepdims=True))
    pooled = jnp.concatenate(rows, axis=0)
    g = jnp.dot(_bf(pooled), _bf(gw1_ref[...].T),
                preferred_element_type=jnp.float32)
    g = jax.nn.relu(_ln(g + gb1_ref[...], gg1_ref[...], gbe1_ref[...]))
    g = jnp.dot(_bf(g), _bf(gw2_ref[...].T),
                preferred_element_type=jnp.float32)
    g = jax.nn.relu(_ln(g + gb2_ref[...], gg2_ref[...], gbe2_ref[...]))
    g_ref[...] = g


def _pool_glob(x1, x2, x3, glob):
    gw1, gb1, gg1, gbe1, gw2, gb2, gg2, gbe2 = glob
    return pl.pallas_call(
        _pool_glob_body,
        out_shape=jax.ShapeDtypeStruct((B, 1024), jnp.float32),
    )(x1, x2, x3, gw1, gb1.reshape(1, -1), gg1.reshape(1, -1),
      gbe1.reshape(1, -1), gw2, gb2.reshape(1, -1), gg2.reshape(1, -1),
      gbe2.reshape(1, -1))


def _head_body(x1_ref, x2_ref, x3_ref, g_ref,
               hw1_ref, hb1_ref, hg1_ref, hbe1_ref,
               hw2_ref, hb2_ref, hg2_ref, hbe2_ref,
               hw3_ref, hb3_ref, hg3_ref, hbe3_ref,
               arcw_ref, out_ref):
    rb = x1_ref.shape[1]
    gfeat = jnp.broadcast_to(g_ref[0], (rb, 1024))
    comb = jnp.concatenate([x1_ref[0], x2_ref[0], x3_ref[0], gfeat], axis=1)
    h = jnp.dot(_bf(comb), _bf(hw1_ref[...].T),
                preferred_element_type=jnp.float32)
    h = jax.nn.relu(_ln(h + hb1_ref[...], hg1_ref[...], hbe1_ref[...]))
    h = jnp.dot(_bf(h), _bf(hw2_ref[...].T),
                preferred_element_type=jnp.float32)
    h = jax.nn.relu(_ln(h + hb2_ref[...], hg2_ref[...], hbe2_ref[...]))
    h = jnp.dot(_bf(h), _bf(hw3_ref[...].T),
                preferred_element_type=jnp.float32)
    h = _ln(h + hb3_ref[...], hg3_ref[...], hbe3_ref[...])
    n = jnp.sqrt(jnp.sum(h * h, axis=1, keepdims=True))
    emb = h / jnp.clip(n, 1e-12, None)
    aw = arcw_ref[...]
    awn = aw / jnp.clip(jnp.sqrt(jnp.sum(aw * aw, axis=1, keepdims=True)),
                        1e-12, None)
    cos = jnp.clip(jnp.dot(_bf(emb), _bf(awn.T),
                           preferred_element_type=jnp.float32), -1.0, 1.0)
    out_ref[0] = cos * 30.0


def _head(x1, x2, x3, g, head, arc_w):
    (hw1, hb1, hg1, hbe1, hw2, hb2, hg2, hbe2, hw3, hb3, hg3, hbe3) = head
    RB = 256
    full = lambda s: pl.BlockSpec(s, lambda b, r: (0, 0))
    return pl.pallas_call(
        _head_body,
        grid=(B, PP // RB),
        compiler_params=pltpu.CompilerParams(
            dimension_semantics=("parallel", "parallel")),
        in_specs=[
            pl.BlockSpec((1, RB, 64), lambda b, r: (b, r, 0)),
            pl.BlockSpec((1, RB, 64), lambda b, r: (b, r, 0)),
            pl.BlockSpec((1, RB, 128), lambda b, r: (b, r, 0)),
            pl.BlockSpec((1, 1, 1024), lambda b, r: (b, 0, 0)),
            full((512, 1280)), full((1, 512)), full((1, 512)), full((1, 512)),
            full((256, 512)), full((1, 256)), full((1, 256)), full((1, 256)),
            full((128, 256)), full((1, 128)), full((1, 128)), full((1, 128)),
            full((3, 128)),
        ],
        out_specs=pl.BlockSpec((1, RB, 3), lambda b, r: (b, r, 0)),
        out_shape=jax.ShapeDtypeStruct((B, PP, 3), jnp.float32),
    )(x1, x2, x3, g,
      hw1, hb1.reshape(1, -1), hg1.reshape(1, -1), hbe1.reshape(1, -1),
      hw2, hb2.reshape(1, -1), hg2.reshape(1, -1), hbe2.reshape(1, -1),
      hw3, hb3.reshape(1, -1), hg3.reshape(1, -1), hbe3.reshape(1, -1),
      arc_w)


def kernel(x, batch, conv1, conv2, conv3, glob, head, arc_w):
    xg = jnp.pad(x.reshape(B, P, 6), ((0, 0), (0, PP - P), (0, 10)))
    w1 = conv1[0]
    w1pad = jnp.concatenate([
        w1[:, :6], jnp.zeros((64, 10), jnp.float32),
        w1[:, 6:], jnp.zeros((64, 10), jnp.float32)], axis=1)
    x1 = _edge_conv(xg, conv1, 16, 64, w1pad=w1pad)
    x2 = _edge_conv(x1, conv2, 64, 64)
    x3 = _edge_conv(x2, conv3, 64, 128)
    g = _pool_glob(x1, x2, x3, glob).reshape(B, 1, 1024)
    out = _head(x1, x2, x3, g, head, arc_w)
    return out[:, :P, :].reshape(B * P, 3)


# speedup vs baseline: 1.0106x; 1.0106x over previous
import functools

import jax
import jax.numpy as jnp
from jax import lax
from jax.experimental import pallas as pl
from jax.experimental.pallas import tpu as pltpu
from jax.experimental.pallas import tpu_sc as plsc

B = 8
P = 1250
PP = 1280
NP = B * PP
R = 128
K = 20
BIG = 1e30


def _ln(x, g, b):
    mu = jnp.mean(x, axis=-1, keepdims=True)
    v = jnp.mean((x - mu) ** 2, axis=-1, keepdims=True)
    return (x - mu) / jnp.sqrt(v + 1e-5) * g + b


def _bf(a):
    return a.astype(jnp.bfloat16)


def _knn_body(xg_ref, idx_ref):
    b = pl.program_id(0)
    r = pl.program_id(1)
    X = xg_ref[0]
    Xr = xg_ref[0, pl.ds(r * R, R), :]

    sq = jnp.sum(X * X, axis=1)
    sqr = jnp.sum(Xr * Xr, axis=1)
    cross = jnp.dot(_bf(Xr), _bf(X).T, preferred_element_type=jnp.float32)
    dist = sqr[:, None] - 2.0 * cross + sq[None, :]
    col = lax.broadcasted_iota(jnp.int32, (R, PP), 1)
    dist = jnp.where(col >= P, BIG, dist)

    base = b * PP
    for k in range(K):
        m = jnp.min(dist, axis=1, keepdims=True)
        hot = dist == m
        am = jnp.min(jnp.where(hot, col, PP), axis=1, keepdims=True)
        dist = jnp.where(col == am, BIG, dist)
        idx_ref[k, 0] = am + base


def _knn(xg, d):
    return pl.pallas_call(
        _knn_body,
        grid=(B, PP // R),
        compiler_params=pltpu.CompilerParams(
            dimension_semantics=("parallel", "parallel")),
        in_specs=[pl.BlockSpec((1, PP, d), lambda b, r: (b, 0, 0))],
        out_specs=pl.BlockSpec((K, 1, R, 1), lambda b, r: (0, b, r, 0)),
        out_shape=jax.ShapeDtypeStruct((K, B, PP, 1), jnp.int32),
    )(xg)


def _sc_gather(table, idx, d):
    info = plsc.get_sparse_core_info()
    nw = info.num_cores * info.num_subcores
    tot = K * NP
    b_per_w = tot // nw
    ch = 800
    n_ch = b_per_w // ch
    mesh = plsc.VectorSubcoreMesh(core_axis_name="c", subcore_axis_name="s")

    @functools.partial(
        pl.kernel, mesh=mesh,
        out_type=jax.ShapeDtypeStruct((tot, d), jnp.float32),
        compiler_params=pltpu.CompilerParams(use_tc_tiling_on_sc=False),
        scratch_types=[
            pltpu.VMEM((b_per_w,), jnp.int32),
            pltpu.VMEM((ch, d), jnp.float32),
            pltpu.VMEM((ch, d), jnp.float32),
            pltpu.SemaphoreType.DMA,
            pltpu.SemaphoreType.DMA,
            pltpu.SemaphoreType.DMA,
            pltpu.SemaphoreType.DMA,
        ],
    )
    def gk(table_hbm, idx_hbm, out_hbm, idx_v, rows0, rows1,
           gs0, gs1, ws0, ws1):
        wid = lax.axis_index("s") * info.num_cores + lax.axis_index("c")
        base = wid * b_per_w
        pltpu.sync_copy(idx_hbm.at[pl.ds(base, b_per_w)], idx_v)
        bufs = (rows0, rows1)
        gsems = (gs0, gs1)
        wsems = (ws0, ws1)
        hg = {}
        hw = {}
        hg[0] = pltpu.async_copy(table_hbm.at[idx_v.at[pl.ds(0, ch)]],
                                 bufs[0], gsems[0])
        for c in range(n_ch):
            cur = c % 2
            if c + 1 < n_ch:
                nxt = (c + 1) % 2
                if c - 1 >= 0:
                    hw[c - 1].wait()
                hg[c + 1] = pltpu.async_copy(
                    table_hbm.at[idx_v.at[pl.ds((c + 1) * ch, ch)]],
                    bufs[nxt], gsems[nxt])
            hg[c].wait()
            hw[c] = pltpu.async_copy(
                bufs[cur], out_hbm.at[pl.ds(base + c * ch, ch)], wsems[cur])
        hw[n_ch - 2].wait()
        hw[n_ch - 1].wait()

    return gk(table, idx)


def _edge_mlp_body(xg_ref, xj_ref, w1_ref, b1_ref, g1_ref, be1_ref,
                   w2_ref, b2_ref, g2_ref, be2_ref, out_ref):
    r = pl.program_id(1)
    xr = xg_ref[0]
    xjs = xj_ref[:, 0]

    xi = jnp.concatenate([xr] * K, axis=0)
    xj = jnp.concatenate([xjs[k] for k in range(K)], axis=0)
    msg = jnp.concatenate([xi, xj - xi], axis=1)

    h = jnp.dot(_bf(msg), _bf(w1_ref[...].T),
                preferred_element_type=jnp.float32) + b1_ref[...]
    h = jax.nn.relu(_ln(h, g1_ref[...], be1_ref[...]))
    h = jnp.dot(_bf(h), _bf(w2_ref[...].T),
                preferred_element_type=jnp.float32) + b2_ref[...]
    h = jax.nn.relu(_ln(h, g2_ref[...], be2_ref[...]))

    H = h.shape[1]
    acc = h[0:R]
    for k in range(1, K):
        acc = jnp.maximum(acc, h[k * R:(k + 1) * R])

    rowid = lax.broadcasted_iota(jnp.int32, (R, H), 0) + r * R
    out_ref[0] = jnp.where(rowid < P, acc, 0.0)


def _edge_mlp(xg, xj, p, d, h):
    w1, b1, g1, be1, w2, b2, g2, be2 = p
    full = lambda s: pl.BlockSpec(s, lambda b, r: (0, 0))
    return pl.pallas_call(
        _edge_mlp_body,
        grid=(B, PP // R),
        compiler_params=pltpu.CompilerParams(
            dimension_semantics=("parallel", "parallel")),
        in_specs=[
            pl.BlockSpec((1, R, d), lambda b, r: (b, r, 0)),
            pl.BlockSpec((K, 1, R, d), lambda b, r: (0, b, r, 0)),
            full((h, w1.shape[1])),
            full((1, h)), full((1, h)), full((1, h)),
            full((h, h)),
            full((1, h)), full((1, h)), full((1, h)),
        ],
        out_specs=pl.BlockSpec((1, R, h), lambda b, r: (b, r, 0)),
        out_shape=jax.ShapeDtypeStruct((B, PP, h), jnp.float32),
    )(xg, xj, w1,
      b1.reshape(1, h), g1.reshape(1, h), be1.reshape(1, h),
      w2, b2.reshape(1, h), g2.reshape(1, h), be2.reshape(1, h))


def _edge_conv(xg, p, d, h, w1pad=None):
    idx = _knn(xg, d)
    rows = _sc_gather(xg.reshape(NP, d), idx.reshape(K * NP), d)
    xj = rows.reshape(K, B, PP, d)
    w1 = p[0] if w1pad is None else w1pad
    return _edge_mlp(xg, xj, (w1,) + tuple(p[1:]), d, h)


def _pool_glob_body(x1_ref, x2_ref, x3_ref,
                    gw1_ref, gb1_ref, gg1_ref, gbe1_ref,
                    gw2_ref, gb2_ref, gg2_ref, gbe2_ref, g_ref):
    rows = []
    for b in range(B):
        loc = jnp.concatenate([x1_ref[b], x2_ref[b], x3_ref[b]], axis=1)
        rows.append(jnp.max(loc, axis=0, keepdims=True))
    pooled = jnp.concatenate(rows, axis=0)
    g = jnp.dot(_bf(pooled), _bf(gw1_ref[...].T),
                preferred_element_type=jnp.float32)
    g = jax.nn.relu(_ln(g + gb1_ref[...], gg1_ref[...], gbe1_ref[...]))
    g = jnp.dot(_bf(g), _bf(gw2_ref[...].T),
                preferred_element_type=jnp.float32)
    g = jax.nn.relu(_ln(g + gb2_ref[...], gg2_ref[...], gbe2_ref[...]))
    g_ref[...] = g


def _pool_glob(x1, x2, x3, glob):
    gw1, gb1, gg1, gbe1, gw2, gb2, gg2, gbe2 = glob
    return pl.pallas_call(
        _pool_glob_body,
        out_shape=jax.ShapeDtypeStruct((B, 1024), jnp.float32),
    )(x1, x2, x3, gw1, gb1.reshape(1, -1), gg1.reshape(1, -1),
      gbe1.reshape(1, -1), gw2, gb2.reshape(1, -1), gg2.reshape(1, -1),
      gbe2.reshape(1, -1))


def _head_body(x1_ref, x2_ref, x3_ref, g_ref,
               hw1_ref, hb1_ref, hg1_ref, hbe1_ref,
               hw2_ref, hb2_ref, hg2_ref, hbe2_ref,
               hw3_ref, hb3_ref, hg3_ref, hbe3_ref,
               arcw_ref, out_ref):
    rb = x1_ref.shape[1]
    gfeat = jnp.broadcast_to(g_ref[0], (rb, 1024))
    comb = jnp.concatenate([x1_ref[0], x2_ref[0], x3_ref[0], gfeat], axis=1)
    h = jnp.dot(_bf(comb), _bf(hw1_ref[...].T),
                preferred_element_type=jnp.float32)
    h = jax.nn.relu(_ln(h + hb1_ref[...], hg1_ref[...], hbe1_ref[...]))
    h = jnp.dot(_bf(h), _bf(hw2_ref[...].T),
                preferred_element_type=jnp.float32)
    h = jax.nn.relu(_ln(h + hb2_ref[...], hg2_ref[...], hbe2_ref[...]))
    h = jnp.dot(_bf(h), _bf(hw3_ref[...].T),
                preferred_element_type=jnp.float32)
    h = _ln(h + hb3_ref[...], hg3_ref[...], hbe3_ref[...])
    n = jnp.sqrt(jnp.sum(h * h, axis=1, keepdims=True))
    emb = h / jnp.clip(n, 1e-12, None)
    aw = arcw_ref[...]
    awn = aw / jnp.clip(jnp.sqrt(jnp.sum(aw * aw, axis=1, keepdims=True)),
                        1e-12, None)
    cos = jnp.clip(jnp.dot(_bf(emb), _bf(awn.T),
                           preferred_element_type=jnp.float32), -1.0, 1.0)
    out_ref[0] = cos * 30.0


def _head(x1, x2, x3, g, head, arc_w):
    (hw1, hb1, hg1, hbe1, hw2, hb2, hg2, hbe2, hw3, hb3, hg3, hbe3) = head
    RB = 256
    full = lambda s: pl.BlockSpec(s, lambda b, r: (0, 0))
    return pl.pallas_call(
        _head_body,
        grid=(B, PP // RB),
        compiler_params=pltpu.CompilerParams(
            dimension_semantics=("parallel", "parallel")),
        in_specs=[
            pl.BlockSpec((1, RB, 64), lambda b, r: (b, r, 0)),
            pl.BlockSpec((1, RB, 64), lambda b, r: (b, r, 0)),
            pl.BlockSpec((1, RB, 128), lambda b, r: (b, r, 0)),
            pl.BlockSpec((1, 1, 1024), lambda b, r: (b, 0, 0)),
            full((512, 1280)), full((1, 512)), full((1, 512)), full((1, 512)),
            full((256, 512)), full((1, 256)), full((1, 256)), full((1, 256)),
            full((128, 256)), full((1, 128)), full((1, 128)), full((1, 128)),
            full((3, 128)),
        ],
        out_specs=pl.BlockSpec((1, RB, 3), lambda b, r: (b, r, 0)),
        out_shape=jax.ShapeDtypeStruct((B, PP, 3), jnp.float32),
    )(x1, x2, x3, g,
      hw1, hb1.reshape(1, -1), hg1.reshape(1, -1), hbe1.reshape(1, -1),
      hw2, hb2.reshape(1, -1), hg2.reshape(1, -1), hbe2.reshape(1, -1),
      hw3, hb3.reshape(1, -1), hg3.reshape(1, -1), hbe3.reshape(1, -1),
      arc_w)


def kernel(x, batch, conv1, conv2, conv3, glob, head, arc_w):
    xg = jnp.pad(x.reshape(B, P, 6), ((0, 0), (0, PP - P), (0, 10)))
    w1 = conv1[0]
    w1pad = jnp.concatenate([
        w1[:, :6], jnp.zeros((64, 10), jnp.float32),
        w1[:, 6:], jnp.zeros((64, 10), jnp.float32)], axis=1)
    x1 = _edge_conv(xg, conv1, 16, 64, w1pad=w1pad)
    x2 = _edge_conv(x1, conv2, 64, 64)
    x3 = _edge_conv(x2, conv3, 64, 128)
    g = _pool_glob(x1, x2, x3, glob).reshape(B, 1, 1024)
    out = _head(x1, x2, x3, g, head, arc_w)
    return out[:, :P, :].reshape(B * P, 3)


# speedup vs baseline: 1.2071x; 1.1944x over previous
import functools

import jax
import jax.numpy as jnp
from jax import lax
from jax.experimental import pallas as pl
from jax.experimental.pallas import tpu as pltpu
from jax.experimental.pallas import tpu_sc as plsc

B = 8
P = 1250
PP = 1280
NP = B * PP
R = 128
K = 20
BIG = 1e30


def _ln(x, g, b):
    mu = jnp.mean(x, axis=-1, keepdims=True)
    v = jnp.mean((x - mu) ** 2, axis=-1, keepdims=True)
    return (x - mu) / jnp.sqrt(v + 1e-5) * g + b


def _bf(a):
    return a.astype(jnp.bfloat16)


def _knn_body(xg_ref, idx_ref):
    b = pl.program_id(0)
    r = pl.program_id(1)
    X = xg_ref[0]
    Xr = xg_ref[0, pl.ds(r * R, R), :]

    sq = jnp.sum(X * X, axis=1)
    sqr = jnp.sum(Xr * Xr, axis=1)
    cross = jnp.dot(_bf(Xr), _bf(X).T, preferred_element_type=jnp.float32)
    dist = sqr[:, None] - 2.0 * cross + sq[None, :]
    colf = lax.broadcasted_iota(jnp.int32, (R, PP), 1).astype(jnp.float32)
    dist = jnp.where(colf >= P, BIG, dist)

    base = b * PP
    for k in range(K):
        m = jnp.min(dist, axis=1, keepdims=True)
        amf = jnp.min(jnp.where(dist == m, colf, 2.0 * PP),
                      axis=1, keepdims=True)
        dist = jnp.where(colf == amf, BIG, dist)
        idx_ref[k, 0] = amf.astype(jnp.int32) + base


def _knn(xg, d):
    return pl.pallas_call(
        _knn_body,
        grid=(B, PP // R),
        compiler_params=pltpu.CompilerParams(
            dimension_semantics=("parallel", "parallel")),
        in_specs=[pl.BlockSpec((1, PP, d), lambda b, r: (b, 0, 0))],
        out_specs=pl.BlockSpec((K, 1, R, 1), lambda b, r: (0, b, r, 0)),
        out_shape=jax.ShapeDtypeStruct((K, B, PP, 1), jnp.int32),
    )(xg)


def _sc_gather(table, idx, d):
    info = plsc.get_sparse_core_info()
    nw = info.num_cores * info.num_subcores
    tot = K * NP
    b_per_w = tot // nw
    ch = 800
    n_ch = b_per_w // ch
    mesh = plsc.VectorSubcoreMesh(core_axis_name="c", subcore_axis_name="s")

    @functools.partial(
        pl.kernel, mesh=mesh,
        out_type=jax.ShapeDtypeStruct((tot, d), jnp.float32),
        compiler_params=pltpu.CompilerParams(use_tc_tiling_on_sc=False),
        scratch_types=[
            pltpu.VMEM((b_per_w,), jnp.int32),
            pltpu.VMEM((ch, d), jnp.float32),
            pltpu.VMEM((ch, d), jnp.float32),
            pltpu.SemaphoreType.DMA,
            pltpu.SemaphoreType.DMA,
            pltpu.SemaphoreType.DMA,
            pltpu.SemaphoreType.DMA,
        ],
    )
    def gk(table_hbm, idx_hbm, out_hbm, idx_v, rows0, rows1,
           gs0, gs1, ws0, ws1):
        wid = lax.axis_index("s") * info.num_cores + lax.axis_index("c")
        base = wid * b_per_w
        pltpu.sync_copy(idx_hbm.at[pl.ds(base, b_per_w)], idx_v)
        bufs = (rows0, rows1)
        gsems = (gs0, gs1)
        wsems = (ws0, ws1)
        hg = {}
        hw = {}
        hg[0] = pltpu.async_copy(table_hbm.at[idx_v.at[pl.ds(0, ch)]],
                                 bufs[0], gsems[0])
        for c in range(n_ch):
            cur = c % 2
            if c + 1 < n_ch:
                nxt = (c + 1) % 2
                if c - 1 >= 0:
                    hw[c - 1].wait()
                hg[c + 1] = pltpu.async_copy(
                    table_hbm.at[idx_v.at[pl.ds((c + 1) * ch, ch)]],
                    bufs[nxt], gsems[nxt])
            hg[c].wait()
            hw[c] = pltpu.async_copy(
                bufs[cur], out_hbm.at[pl.ds(base + c * ch, ch)], wsems[cur])
        hw[n_ch - 2].wait()
        hw[n_ch - 1].wait()

    return gk(table, idx)


def _edge_mlp_body(xg_ref, xj_ref, w1_ref, b1_ref, g1_ref, be1_ref,
                   w2_ref, b2_ref, g2_ref, be2_ref, out_ref):
    r = pl.program_id(1)
    xr = xg_ref[0]
    xjs = xj_ref[:, 0]

    xi = jnp.concatenate([xr] * K, axis=0)
    xj = jnp.concatenate([xjs[k] for k in range(K)], axis=0)
    msg = jnp.concatenate([xi, xj - xi], axis=1)

    h = jnp.dot(_bf(msg), _bf(w1_ref[...].T),
                preferred_element_type=jnp.float32) + b1_ref[...]
    h = jax.nn.relu(_ln(h, g1_ref[...], be1_ref[...]))
    h = jnp.dot(_bf(h), _bf(w2_ref[...].T),
                preferred_element_type=jnp.float32) + b2_ref[...]
    h = jax.nn.relu(_ln(h, g2_ref[...], be2_ref[...]))

    H = h.shape[1]
    acc = h[0:R]
    for k in range(1, K):
        acc = jnp.maximum(acc, h[k * R:(k + 1) * R])

    rowid = lax.broadcasted_iota(jnp.int32, (R, H), 0) + r * R
    out_ref[0] = jnp.where(rowid < P, acc, 0.0)


def _edge_mlp(xg, xj, p, d, h):
    w1, b1, g1, be1, w2, b2, g2, be2 = p
    full = lambda s: pl.BlockSpec(s, lambda b, r: (0, 0))
    return pl.pallas_call(
        _edge_mlp_body,
        grid=(B, PP // R),
        compiler_params=pltpu.CompilerParams(
            dimension_semantics=("parallel", "parallel")),
        in_specs=[
            pl.BlockSpec((1, R, d), lambda b, r: (b, r, 0)),
            pl.BlockSpec((K, 1, R, d), lambda b, r: (0, b, r, 0)),
            full((h, w1.shape[1])),
            full((1, h)), full((1, h)), full((1, h)),
            full((h, h)),
            full((1, h)), full((1, h)), full((1, h)),
        ],
        out_specs=pl.BlockSpec((1, R, h), lambda b, r: (b, r, 0)),
        out_shape=jax.ShapeDtypeStruct((B, PP, h), jnp.float32),
    )(xg, xj, w1,
      b1.reshape(1, h), g1.reshape(1, h), be1.reshape(1, h),
      w2, b2.reshape(1, h), g2.reshape(1, h), be2.reshape(1, h))


def _edge_conv(xg, p, d, h, w1pad=None):
    idx = _knn(xg, d)
    rows = _sc_gather(xg.reshape(NP, d), idx.reshape(K * NP), d)
    xj = rows.reshape(K, B, PP, d)
    w1 = p[0] if w1pad is None else w1pad
    return _edge_mlp(xg, xj, (w1,) + tuple(p[1:]), d, h)


def _pool_glob_body(x1_ref, x2_ref, x3_ref,
                    gw1_ref, gb1_ref, gg1_ref, gbe1_ref,
                    gw2_ref, gb2_ref, gg2_ref, gbe2_ref, g_ref):
    rows = []
    for b in range(B):
        loc = jnp.concatenate([x1_ref[b], x2_ref[b], x3_ref[b]], axis=1)
        rows.append(jnp.max(loc, axis=0, keepdims=True))
    pooled = jnp.concatenate(rows, axis=0)
    g = jnp.dot(_bf(pooled), _bf(gw1_ref[...].T),
                preferred_element_type=jnp.float32)
    g = jax.nn.relu(_ln(g + gb1_ref[...], gg1_ref[...], gbe1_ref[...]))
    g = jnp.dot(_bf(g), _bf(gw2_ref[...].T),
                preferred_element_type=jnp.float32)
    g = jax.nn.relu(_ln(g + gb2_ref[...], gg2_ref[...], gbe2_ref[...]))
    g_ref[...] = g


def _pool_glob(x1, x2, x3, glob):
    gw1, gb1, gg1, gbe1, gw2, gb2, gg2, gbe2 = glob
    return pl.pallas_call(
        _pool_glob_body,
        out_shape=jax.ShapeDtypeStruct((B, 1024), jnp.float32),
    )(x1, x2, x3, gw1, gb1.reshape(1, -1), gg1.reshape(1, -1),
      gbe1.reshape(1, -1), gw2, gb2.reshape(1, -1), gg2.reshape(1, -1),
      gbe2.reshape(1, -1))


def _head_body(x1_ref, x2_ref, x3_ref, g_ref,
               hw1_ref, hb1_ref, hg1_ref, hbe1_ref,
               hw2_ref, hb2_ref, hg2_ref, hbe2_ref,
               hw3_ref, hb3_ref, hg3_ref, hbe3_ref,
               arcw_ref, out_ref):
    rb = x1_ref.shape[1]
    gfeat = jnp.broadcast_to(g_ref[0], (rb, 1024))
    comb = jnp.concatenate([x1_ref[0], x2_ref[0], x3_ref[0], gfeat], axis=1)
    h = jnp.dot(_bf(comb), _bf(hw1_ref[...].T),
                preferred_element_type=jnp.float32)
    h = jax.nn.relu(_ln(h + hb1_ref[...], hg1_ref[...], hbe1_ref[...]))
    h = jnp.dot(_bf(h), _bf(hw2_ref[...].T),
                preferred_element_type=jnp.float32)
    h = jax.nn.relu(_ln(h + hb2_ref[...], hg2_ref[...], hbe2_ref[...]))
    h = jnp.dot(_bf(h), _bf(hw3_ref[...].T),
                preferred_element_type=jnp.float32)
    h = _ln(h + hb3_ref[...], hg3_ref[...], hbe3_ref[...])
    n = jnp.sqrt(jnp.sum(h * h, axis=1, keepdims=True))
    emb = h / jnp.clip(n, 1e-12, None)
    aw = arcw_ref[...]
    awn = aw / jnp.clip(jnp.sqrt(jnp.sum(aw * aw, axis=1, keepdims=True)),
                        1e-12, None)
    cos = jnp.clip(jnp.dot(_bf(emb), _bf(awn.T),
                           preferred_element_type=jnp.float32), -1.0, 1.0)
    out_ref[0] = cos * 30.0


def _head(x1, x2, x3, g, head, arc_w):
    (hw1, hb1, hg1, hbe1, hw2, hb2, hg2, hbe2, hw3, hb3, hg3, hbe3) = head
    RB = 256
    full = lambda s: pl.BlockSpec(s, lambda b, r: (0, 0))
    return pl.pallas_call(
        _head_body,
        grid=(B, PP // RB),
        compiler_params=pltpu.CompilerParams(
            dimension_semantics=("parallel", "parallel")),
        in_specs=[
            pl.BlockSpec((1, RB, 64), lambda b, r: (b, r, 0)),
            pl.BlockSpec((1, RB, 64), lambda b, r: (b, r, 0)),
            pl.BlockSpec((1, RB, 128), lambda b, r: (b, r, 0)),
            pl.BlockSpec((1, 1, 1024), lambda b, r: (b, 0, 0)),
            full((512, 1280)), full((1, 512)), full((1, 512)), full((1, 512)),
            full((256, 512)), full((1, 256)), full((1, 256)), full((1, 256)),
            full((128, 256)), full((1, 128)), full((1, 128)), full((1, 128)),
            full((3, 128)),
        ],
        out_specs=pl.BlockSpec((1, RB, 3), lambda b, r: (b, r, 0)),
        out_shape=jax.ShapeDtypeStruct((B, PP, 3), jnp.float32),
    )(x1, x2, x3, g,
      hw1, hb1.reshape(1, -1), hg1.reshape(1, -1), hbe1.reshape(1, -1),
      hw2, hb2.reshape(1, -1), hg2.reshape(1, -1), hbe2.reshape(1, -1),
      hw3, hb3.reshape(1, -1), hg3.reshape(1, -1), hbe3.reshape(1, -1),
      arc_w)


def kernel(x, batch, conv1, conv2, conv3, glob, head, arc_w):
    xg = jnp.pad(x.reshape(B, P, 6), ((0, 0), (0, PP - P), (0, 10)))
    w1 = conv1[0]
    w1pad = jnp.concatenate([
        w1[:, :6], jnp.zeros((64, 10), jnp.float32),
        w1[:, 6:], jnp.zeros((64, 10), jnp.float32)], axis=1)
    x1 = _edge_conv(xg, conv1, 16, 64, w1pad=w1pad)
    x2 = _edge_conv(x1, conv2, 64, 64)
    x3 = _edge_conv(x2, conv3, 64, 128)
    g = _pool_glob(x1, x2, x3, glob).reshape(B, 1, 1024)
    out = _head(x1, x2, x3, g, head, arc_w)
    return out[:, :P, :].reshape(B * P, 3)
